# Initial kernel scaffold; baseline (speedup 1.0000x reference)
import jax
import jax.numpy as jnp
from jax.experimental import pallas as pl


def kernel(logits, labels):
    raise NotImplementedError("write your pallas kernel here")


# speedup vs baseline: 9.2705x; 9.2705x over previous
import functools

import jax
import jax.numpy as jnp
from jax import lax
from jax.experimental import pallas as pl
from jax.experimental.pallas import tpu as pltpu
from jax.experimental.pallas import tpu_sc as plsc

NC = 2
NS = 16
L = 16
NW = NC * NS
NBINS = 10
NSLOT = NBINS + 1
CH = 16384
UN = 8


def _sc_partials(x, y):
    n = x.shape[0]
    per_w = n // NW
    n_chunks = per_w // CH
    mesh = plsc.VectorSubcoreMesh(
        core_axis_name="c", subcore_axis_name="s", num_cores=NC, num_subcores=NS
    )

    @functools.partial(
        pl.kernel,
        out_type=jax.ShapeDtypeStruct((NW, NSLOT * L), jnp.float32),
        mesh=mesh,
        scratch_types=[
            pltpu.VMEM((2, CH), jnp.float32),
            pltpu.VMEM((2, CH), jnp.int32),
            pltpu.VMEM((NSLOT * L,), jnp.float32),
            pltpu.SemaphoreType.DMA,
            pltpu.SemaphoreType.DMA,
            pltpu.SemaphoreType.DMA,
            pltpu.SemaphoreType.DMA,
        ],
        compiler_params=pltpu.CompilerParams(
            needs_layout_passes=False, use_tc_tiling_on_sc=False
        ),
    )
    def body(x_hbm, y_hbm, out_hbm, xbuf, ybuf, acc, sx0, sx1, sy0, sy1):
        sems = ((sx0, sy0), (sx1, sy1))
        wid = lax.axis_index("s") * NC + lax.axis_index("c")
        base = wid * per_w
        for k in range(NSLOT):
            acc[pl.ds(k * L, L)] = jnp.zeros((L,), jnp.float32)
        lanes = lax.iota(jnp.int32, L)

        def start(ci, b):
            off = base + ci * CH
            pltpu.async_copy(x_hbm.at[pl.ds(off, CH)], xbuf.at[b], sems[b][0])
            pltpu.async_copy(y_hbm.at[pl.ds(off, CH)], ybuf.at[b], sems[b][1])

        def wait(b):
            pltpu.make_async_copy(x_hbm.at[pl.ds(0, CH)], xbuf.at[b], sems[b][0]).wait()
            pltpu.make_async_copy(y_hbm.at[pl.ds(0, CH)], ybuf.at[b], sems[b][1]).wait()

        start(0, 0)

        def chunk2(co, carry):
            for b in range(2):
                ci = co * 2 + b

                @pl.when(ci + 1 < n_chunks)
                def _():
                    start(ci + 1, b ^ 1)

                wait(b)

                def vec_body(vi, c2):
                    o0 = vi * (L * UN)
                    xs = [xbuf[b, pl.ds(o0 + u * L, L)] for u in range(UN)]
                    ys = [ybuf[b, pl.ds(o0 + u * L, L)] for u in range(UN)]
                    es = [jnp.exp(-xv) for xv in xs]
                    ps = [1.0 / (1.0 + e) for e in es]
                    for u in range(UN):
                        p = ps[u]
                        bb = (p * 10.0).astype(jnp.int32)
                        yf = ys[u].astype(jnp.float32)
                        av = jnp.where(xs[u] > 0.0, yf, 1.0 - yf)
                        plsc.addupdate_scatter(acc, [bb * L + lanes], p - av)
                    return c2

                lax.fori_loop(0, CH // (L * UN), vec_body, 0)
            return carry

        lax.fori_loop(0, n_chunks // 2, chunk2, 0)
        pltpu.sync_copy(acc, out_hbm.at[wid])

    return body(x, y)


def _combine(part_ref, out_ref, *, inv_n):
    colsum = jnp.sum(part_ref[...], axis=0, keepdims=True)
    grp = lax.broadcasted_iota(jnp.int32, (1, NSLOT * L), 1) // L
    d = []
    for k in range(NSLOT):
        d.append(jnp.sum(jnp.where(grp == k, colsum, 0.0)))
    total = jnp.abs(d[NBINS - 1] + d[NBINS])
    for k in range(NBINS - 1):
        total = total + jnp.abs(d[k])
    out_ref[...] = jnp.full((1, 1), total * inv_n, jnp.float32)


def kernel(logits, labels):
    n = logits.shape[0]
    x = logits.reshape(n)
    y = labels.reshape(n)
    partials = _sc_partials(x, y)
    ece = pl.pallas_call(
        functools.partial(_combine, inv_n=1.0 / float(n)),
        out_shape=jax.ShapeDtypeStruct((1, 1), jnp.float32),
    )(partials)
    return ece.reshape(1)
